# Initial kernel scaffold; baseline (speedup 1.0000x reference)
import jax
import jax.numpy as jnp
from jax.experimental import pallas as pl


def kernel(input, token_table, pos_table):
    raise NotImplementedError("write your pallas kernel here")


# speedup vs baseline: 2.8723x; 2.8723x over previous
import functools

import jax
import jax.numpy as jnp
from jax import lax
from jax.experimental import pallas as pl
from jax.experimental.pallas import tpu as pltpu
from jax.experimental.pallas import tpu_sc as plsc

_EMB = 128
_WIN = 64
_LANES = 16
_REGS_PER_ROW = _EMB // _LANES


@functools.partial(jax.jit, static_argnames=())
def _run(flat_idx, token_table, pos_table):
    N = flat_idx.shape[0]
    V, D = token_table.shape

    info = plsc.get_sparse_core_info()
    NC, NS = info.num_cores, info.num_subcores
    NW = NC * NS
    n_per_w = N // NW
    CH = 512
    n_ch = n_per_w // CH

    mesh = plsc.VectorSubcoreMesh(core_axis_name="c", subcore_axis_name="s")

    @functools.partial(
        pl.kernel,
        mesh=mesh,
        out_type=jax.ShapeDtypeStruct((N, D), jnp.float32),
        scratch_types=[
            pltpu.VMEM((CH,), jnp.int32),
            pltpu.VMEM((CH, D), jnp.float32),
            pltpu.VMEM((_WIN, D), jnp.float32),
            pltpu.SemaphoreType.DMA,
        ],
    )
    def k(idx_hbm, tok_hbm, pos_hbm, out_hbm, idx_v, rows_v, pos_v, sem):
        wid = lax.axis_index("s") * NC + lax.axis_index("c")
        base = wid * n_per_w
        pltpu.sync_copy(pos_hbm, pos_v)

        def chunk_body(ci, _):
            off = base + ci * CH
            pltpu.sync_copy(idx_hbm.at[pl.ds(off, CH)], idx_v)
            pltpu.async_copy(tok_hbm.at[idx_v], rows_v, sem).wait()

            def w_body(w, _):
                pos_regs = [pos_v[w, pl.ds(kk * _LANES, _LANES)]
                            for kk in range(_REGS_PER_ROW)]

                def r_body(r, _):
                    row = r * _WIN + w
                    for kk in range(_REGS_PER_ROW):
                        sl = pl.ds(kk * _LANES, _LANES)
                        rows_v[row, sl] = rows_v[row, sl] + pos_regs[kk]
                    return 0

                lax.fori_loop(0, CH // _WIN, r_body, 0)
                return 0

            lax.fori_loop(0, _WIN, w_body, 0)
            pltpu.sync_copy(rows_v, out_hbm.at[pl.ds(off, CH)])
            return 0

        lax.fori_loop(0, n_ch, chunk_body, 0)

    return k(flat_idx, token_table, pos_table)


def kernel(input, token_table, pos_table):
    B, W = input.shape
    D = token_table.shape[1]
    flat_idx = input.reshape(B * W).astype(jnp.int32)
    out = _run(flat_idx, token_table, pos_table)
    return out.reshape(B, W, D)


# speedup vs baseline: 6.8626x; 2.3892x over previous
import functools

import jax
import jax.numpy as jnp
from jax import lax
from jax.experimental import pallas as pl
from jax.experimental.pallas import tpu as pltpu
from jax.experimental.pallas import tpu_sc as plsc

_EMB = 128
_WIN = 64
_LANES = 16
_REGS_PER_ROW = _EMB // _LANES
_CH = 128
_NBUF = 4


def _run(flat_idx, token_table, pos_table):
    N = flat_idx.shape[0]
    V, D = token_table.shape

    info = plsc.get_sparse_core_info()
    NC, NS = info.num_cores, info.num_subcores
    NW = NC * NS
    n_per_w = N // NW
    n_ch = n_per_w // _CH
    rounds = n_ch // _NBUF

    mesh = plsc.VectorSubcoreMesh(core_axis_name="c", subcore_axis_name="s")

    @functools.partial(
        pl.kernel,
        mesh=mesh,
        out_type=jax.ShapeDtypeStruct((N, D), jnp.float32),
        scratch_types=(
            [pltpu.VMEM((n_per_w,), jnp.int32),
             pltpu.VMEM((_WIN, D), jnp.float32)]
            + [pltpu.VMEM((_CH, D), jnp.float32) for _ in range(_NBUF)]
            + [pltpu.SemaphoreType.DMA for _ in range(2 * _NBUF)]
        ),
    )
    def k(idx_hbm, tok_hbm, pos_hbm, out_hbm, idx_all, pos_v, *bufs_and_sems):
        rows = list(bufs_and_sems[:_NBUF])
        gsem = list(bufs_and_sems[_NBUF:2 * _NBUF])
        osem = list(bufs_and_sems[2 * _NBUF:])

        wid = lax.axis_index("s") * NC + lax.axis_index("c")
        base = wid * n_per_w
        pltpu.sync_copy(pos_hbm, pos_v)
        pltpu.sync_copy(idx_hbm.at[pl.ds(base, n_per_w)], idx_all)

        def gather_copy(lci, b):
            src = tok_hbm.at[idx_all.at[pl.ds(lci * _CH, _CH)]]
            return pltpu.make_async_copy(src, rows[b], gsem[b])

        def out_copy(lci, b):
            return pltpu.make_async_copy(
                rows[b], out_hbm.at[pl.ds(base + lci * _CH, _CH)], osem[b])

        for b in range(_NBUF):
            gather_copy(b, b).start()

        def round_body(i, _):
            for b in range(_NBUF):
                lci = i * _NBUF + b
                gather_copy(lci, b).wait()
                rows_b = rows[b]

                def w_body(w, _):
                    for kk in range(_REGS_PER_ROW):
                        sl = pl.ds(kk * _LANES, _LANES)
                        p = pos_v[w, sl]
                        for r in range(_CH // _WIN):
                            row = r * _WIN + w
                            rows_b[row, sl] = rows_b[row, sl] + p
                    return 0

                lax.fori_loop(0, _WIN, w_body, 0)
                out_copy(lci, b).start()

            @pl.when(i < rounds - 1)
            def _():
                for b in range(_NBUF):
                    lci = i * _NBUF + b
                    out_copy(lci, b).wait()
                    gather_copy(lci + _NBUF, b).start()

            return 0

        lax.fori_loop(0, rounds, round_body, 0)
        for b in range(_NBUF):
            lci = (rounds - 1) * _NBUF + b
            out_copy(lci, b).wait()

    return k(flat_idx, token_table, pos_table)


def kernel(input, token_table, pos_table):
    B, W = input.shape
    D = token_table.shape[1]
    flat_idx = input.reshape(B * W).astype(jnp.int32)
    out = _run(flat_idx, token_table, pos_table)
    return out.reshape(B, W, D)
